# Initial kernel scaffold; baseline (speedup 1.0000x reference)
import jax
import jax.numpy as jnp
from jax.experimental import pallas as pl


def kernel(x, adj, gat1_Wl, gat1_bl, gat1_Wr, gat1_br, gat1_att, gat1_b, bn1_g, bn1_b, gat2_Wl, gat2_bl, gat2_Wr, gat2_br, gat2_att, gat2_b, gat3_Wl, gat3_bl, gat3_Wr, gat3_br, gat3_att, gat3_b, dec_W1, dec_b1, dec_bn_g, dec_bn_b, dec_W2, dec_b2):
    raise NotImplementedError("write your pallas kernel here")


# speedup vs baseline: 6.7494x; 6.7494x over previous
import functools

import jax
import jax.numpy as jnp
from jax import lax
from jax.experimental import pallas as pl
from jax.experimental.pallas import tpu as pltpu
from jax.experimental.pallas import tpu_sc as plsc

N = 10000
NFEAT = 128
NHID1 = 64
NHID2 = 32
H1 = 2 * NHID2
E = 160000
E_REAL = E + N
EDGE_BLOCK = 128
E_PAD = 172032
ROWS_PER_TILE = 624
TAIL_ROWS = N - 16 * ROWS_PER_TILE
TAIL_OFF = 16 * ROWS_PER_TILE
STAGE_ROWS = 48
NCHUNK = ROWS_PER_TILE // STAGE_ROWS
_BN_INV = float(1.0 / (1.0 + 1e-5) ** 0.5)

ROW_BLOCK = 1000
GRID_N = N // ROW_BLOCK


FP = 96
FR = 64
HEADS = ((0, 48, 0, 0, 32), (48, 48, 32, 32, 32))


NRANGE = 312
ACC_ROWS = 344
TRASH = 336


def _make_gat_edges(nc):
    mesh = plsc.VectorSubcoreMesh(core_axis_name="c", subcore_axis_name="s",
                                  num_cores=nc)

    @functools.partial(
        pl.kernel,
        mesh=mesh,
        compiler_params=pltpu.CompilerParams(needs_layout_passes=False,
                                             use_tc_tiling_on_sc=False),
        out_type=jax.ShapeDtypeStruct((N, FP), jnp.float32),
        scratch_types=[
            pltpu.VMEM((EDGE_BLOCK,), jnp.int32),
            pltpu.VMEM((EDGE_BLOCK,), jnp.int32),
            pltpu.VMEM((EDGE_BLOCK, FP), jnp.float32),
            pltpu.VMEM((EDGE_BLOCK, FR), jnp.float32),
            pltpu.VMEM((FR,), jnp.float32),
            pltpu.VMEM((16,), jnp.float32),
            pltpu.VMEM((16, 16), jnp.float32),
            pltpu.VMEM((2, 16), jnp.float32),
            pltpu.VMEM((32, 16), jnp.int32),
            pltpu.VMEM((ACC_ROWS, FP), jnp.float32),
            pltpu.SemaphoreType.DMA,
            pltpu.SemaphoreType.DMA,
        ],
    )
    def kern(xl_hbm, xr_hbm, src_hbm, dst_hbm, att_hbm, cfg_hbm, meta_hbm,
             out_hbm, idx_s, idx_d, gl, gr, att_v, cfg_v, dotbuf, wab,
             meta_v, accbuf, sem1, sem2):
        cid = lax.axis_index("c")
        sid = lax.axis_index("s")
        wid = sid * nc + cid

        pltpu.sync_copy(att_hbm, att_v)
        pltpu.sync_copy(cfg_hbm, cfg_v)
        pltpu.sync_copy(meta_hbm, meta_v)
        att_regs = [att_v[pl.ds(16 * k, 16)] for k in range(FR // 16)]
        cvec = cfg_v[...]
        mv = meta_v[wid, :]
        lo = mv[0]
        hi = mv[1]
        bstart = mv[2]
        nblk = mv[3]

        zero16 = jnp.zeros((16,), jnp.float32)

        def block_body(b, carry):
            accs, d_prev = carry
            base = pl.multiple_of(bstart + b * EDGE_BLOCK, EDGE_BLOCK)
            pltpu.sync_copy(src_hbm.at[pl.ds(base, EDGE_BLOCK)], idx_s)
            pltpu.sync_copy(dst_hbm.at[pl.ds(base, EDGE_BLOCK)], idx_d)
            c1 = pltpu.async_copy(xl_hbm.at[idx_s], gl, sem1)
            c2 = pltpu.async_copy(xr_hbm.at[idx_d], gr, sem2)
            c1.wait()
            c2.wait()

            def t_body(t, tcarry):
                taccs, td_prev = tcarry
                e0 = t * 16
                rows = lax.iota(jnp.int32, 16)
                dvecs = []
                for (GO, GW, RO, AO, F) in HEADS:
                    for e_ in range(16):
                        acc = zero16
                        for k in range(F // 16):
                            a = gl[e0 + e_, pl.ds(GO + 16 * k, 16)]
                            r = gr[e0 + e_, pl.ds(RO + 16 * k, 16)]
                            m = a + r
                            m = jnp.maximum(m, 0.2 * m)
                            acc = acc + m * att_regs[AO // 16 + k]
                        dotbuf[e_, :] = acc
                    tot = zero16
                    for l in range(16):
                        tot = tot + plsc.load_gather(
                            dotbuf, [rows, jnp.full((16,), l, jnp.int32)])
                    dvecs.append(tot)
                wvec_a = jnp.exp(dvecs[0] + cvec * dvecs[1])
                wvec_b = jnp.exp(cvec * dvecs[0] + dvecs[1])
                dvec = idx_d[pl.ds(e0, 16)]
                for e_ in range(16):
                    e = e0 + e_
                    d = dvec[e_]
                    flush = d != td_prev
                    inr = jnp.logical_and(td_prev >= lo, td_prev < hi)
                    strow = jnp.where(jnp.logical_and(flush, inr),
                                      td_prev - lo, TRASH)
                    for j in range(FP // 16):
                        accbuf[strow, pl.ds(16 * j, 16)] = taccs[j]
                    keep = jnp.where(flush, 0.0, 1.0)
                    keepv = jnp.full((16,), keep)
                    wva = jnp.full((16,), wvec_a[e_])
                    wvb = jnp.full((16,), wvec_b[e_])
                    new = []
                    for j in range(FP // 16):
                        wv = wva if (16 * j) < 48 else wvb
                        contrib = gl[e, pl.ds(16 * j, 16)] * wv
                        new.append(taccs[j] * keepv + contrib)
                    taccs = new
                    td_prev = d
                return taccs, td_prev

            return lax.fori_loop(0, EDGE_BLOCK // 16, t_body,
                                 (accs, d_prev))

        accs0 = [zero16] * (FP // 16)
        accs, d_prev = lax.fori_loop(0, nblk, block_body,
                                     (accs0, jnp.int32(-1)))
        inr = jnp.logical_and(d_prev >= lo, d_prev < hi)
        strow = jnp.where(inr, d_prev - lo, TRASH)
        for j in range(FP // 16):
            accbuf[strow, pl.ds(16 * j, 16)] = accs[j]

        pltpu.sync_copy(accbuf.at[pl.ds(0, NRANGE)],
                        out_hbm.at[pl.ds(lo, NRANGE)])

        @pl.when(wid == 31)
        def _tail():
            pltpu.sync_copy(accbuf.at[pl.ds(NRANGE, 16)],
                            out_hbm.at[pl.ds(31 * NRANGE + NRANGE, 16)])

    return kern


NC = 2


@functools.lru_cache(maxsize=None)
def _gat_edges():
    return _make_gat_edges(NC)


def _proj1_body(x_ref, wlt_ref, blaug_ref, wrt_ref, br_ref, xl_out, xr_out):
    xb = x_ref[...]
    xl_out[...] = (jnp.dot(xb, wlt_ref[...], preferred_element_type=jnp.float32)
                   + blaug_ref[...])
    xr_out[...] = (jnp.dot(xb, wrt_ref[...], preferred_element_type=jnp.float32)
                   + br_ref[...])


def _proj1(x, wlt, blaug, wrt, br):
    return pl.pallas_call(
        _proj1_body,
        grid=(GRID_N,),
        in_specs=[
            pl.BlockSpec((ROW_BLOCK, NFEAT), lambda i: (i, 0)),
            pl.BlockSpec((NFEAT, FP), lambda i: (0, 0)),
            pl.BlockSpec((1, FP), lambda i: (0, 0)),
            pl.BlockSpec((NFEAT, H1), lambda i: (0, 0)),
            pl.BlockSpec((1, H1), lambda i: (0, 0)),
        ],
        out_specs=[
            pl.BlockSpec((ROW_BLOCK, FP), lambda i: (i, 0)),
            pl.BlockSpec((ROW_BLOCK, H1), lambda i: (i, 0)),
        ],
        out_shape=[
            jax.ShapeDtypeStruct((N, FP), jnp.float32),
            jax.ShapeDtypeStruct((N, H1), jnp.float32),
        ],
    )(x, wlt, blaug, wrt, br)


def _mid_body(p_ref, bn1g_ref, bn1b_ref, g1b_ref, wl23_ref, bl23_ref,
              wr23_ref, br23_ref, xl_out, xr_out):
    num = p_ref[...]
    den = num[:, 32:33]
    xl1 = jnp.concatenate([num[:, 0:32], num[:, 48:80]], axis=1)
    o = xl1 / (den + 1e-16) + g1b_ref[...]
    h = jnp.maximum(bn1g_ref[...] * (o * _BN_INV) + bn1b_ref[...], 0.0)
    xl_out[...] = (jnp.dot(h, wl23_ref[...], preferred_element_type=jnp.float32)
                   + bl23_ref[...])
    xr_out[...] = (jnp.dot(h, wr23_ref[...], preferred_element_type=jnp.float32)
                   + br23_ref[...])


def _mid(p, bn1g, bn1b, g1b, wl23, bl23, wr23, br23):
    return pl.pallas_call(
        _mid_body,
        grid=(GRID_N,),
        in_specs=[
            pl.BlockSpec((ROW_BLOCK, 96), lambda i: (i, 0)),
            pl.BlockSpec((1, H1), lambda i: (0, 0)),
            pl.BlockSpec((1, H1), lambda i: (0, 0)),
            pl.BlockSpec((1, H1), lambda i: (0, 0)),
            pl.BlockSpec((H1, 96), lambda i: (0, 0)),
            pl.BlockSpec((1, 96), lambda i: (0, 0)),
            pl.BlockSpec((H1, H1), lambda i: (0, 0)),
            pl.BlockSpec((1, H1), lambda i: (0, 0)),
        ],
        out_specs=[
            pl.BlockSpec((ROW_BLOCK, 96), lambda i: (i, 0)),
            pl.BlockSpec((ROW_BLOCK, H1), lambda i: (i, 0)),
        ],
        out_shape=[
            jax.ShapeDtypeStruct((N, 96), jnp.float32),
            jax.ShapeDtypeStruct((N, H1), jnp.float32),
        ],
    )(p, bn1g, bn1b, g1b, wl23, bl23, wr23, br23)


def _dec_body(p_ref, g2b_ref, g3b_ref, w1t_ref, db1_ref, bng_ref, bnb_ref,
              w2t_ref, db2_ref, mu_out, lv_out, rex_out):
    num = p_ref[...]
    mu = num[:, 0:32] / (num[:, 32:33] + 1e-16) + g2b_ref[...]
    lv = num[:, 48:80] / (num[:, 80:81] + 1e-16) + g3b_ref[...]
    mu_out[...] = mu
    lv_out[...] = lv
    d1 = jnp.dot(mu, w1t_ref[...], preferred_element_type=jnp.float32) + db1_ref[...]
    d1 = jnp.maximum(bng_ref[...] * (d1 * _BN_INV) + bnb_ref[...], 0.0)
    rex_out[...] = (jnp.dot(d1, w2t_ref[...], preferred_element_type=jnp.float32)
                    + db2_ref[...])


def _dec(p, g2b, g3b, w1t, db1, bng, bnb, w2t, db2):
    return pl.pallas_call(
        _dec_body,
        grid=(GRID_N,),
        in_specs=[
            pl.BlockSpec((ROW_BLOCK, 96), lambda i: (i, 0)),
            pl.BlockSpec((1, NHID2), lambda i: (0, 0)),
            pl.BlockSpec((1, NHID2), lambda i: (0, 0)),
            pl.BlockSpec((NHID2, NHID1), lambda i: (0, 0)),
            pl.BlockSpec((1, NHID1), lambda i: (0, 0)),
            pl.BlockSpec((1, NHID1), lambda i: (0, 0)),
            pl.BlockSpec((1, NHID1), lambda i: (0, 0)),
            pl.BlockSpec((NHID1, NFEAT), lambda i: (0, 0)),
            pl.BlockSpec((1, NFEAT), lambda i: (0, 0)),
        ],
        out_specs=[
            pl.BlockSpec((ROW_BLOCK, NHID2), lambda i: (i, 0)),
            pl.BlockSpec((ROW_BLOCK, NHID2), lambda i: (i, 0)),
            pl.BlockSpec((ROW_BLOCK, NFEAT), lambda i: (i, 0)),
        ],
        out_shape=[
            jax.ShapeDtypeStruct((N, NHID2), jnp.float32),
            jax.ShapeDtypeStruct((N, NHID2), jnp.float32),
            jax.ShapeDtypeStruct((N, NFEAT), jnp.float32),
        ],
    )(p, g2b, g3b, w1t, db1, bng, bnb, w2t, db2)


def _readj_body(rows_ref, full_ref, out_ref):
    logits = lax.dot_general(
        rows_ref[...], full_ref[...],
        dimension_numbers=(((1,), (1,)), ((), ())),
        preferred_element_type=jnp.float32)
    out_ref[...] = jax.nn.sigmoid(logits)


RJ_BLOCK = 400


def _readj(emb):
    return pl.pallas_call(
        _readj_body,
        grid=(N // RJ_BLOCK,),
        in_specs=[
            pl.BlockSpec((RJ_BLOCK, NHID2), lambda i: (i, 0)),
            pl.BlockSpec((N, NHID2), lambda i: (0, 0)),
        ],
        out_specs=pl.BlockSpec((RJ_BLOCK, N), lambda i: (i, 0)),
        out_shape=jax.ShapeDtypeStruct((N, N), jnp.float32),
    )(emb, emb)


def kernel(x, adj, gat1_Wl, gat1_bl, gat1_Wr, gat1_br, gat1_att, gat1_b,
           bn1_g, bn1_b, gat2_Wl, gat2_bl, gat2_Wr, gat2_br, gat2_att, gat2_b,
           gat3_Wl, gat3_bl, gat3_Wr, gat3_br, gat3_att, gat3_b,
           dec_W1, dec_b1, dec_bn_g, dec_bn_b, dec_W2, dec_b2):
    f32 = jnp.float32

    loop = jnp.arange(N, dtype=jnp.int32)
    src0 = jnp.concatenate(
        [adj[0], loop, jnp.full((E_PAD - E_REAL,), N, jnp.int32)])
    dst0 = jnp.concatenate(
        [adj[1], loop, jnp.zeros((E_PAD - E_REAL,), jnp.int32)])
    order = jnp.argsort(dst0)
    src = src0[order]
    dst = dst0[order]

    wids = jnp.arange(32, dtype=jnp.int32)
    lo = wids * NRANGE
    hi = jnp.where(wids == 31, N, lo + NRANGE).astype(jnp.int32)
    estart = jnp.searchsorted(dst, lo).astype(jnp.int32)
    eend = jnp.searchsorted(dst, hi).astype(jnp.int32)
    bstart = (estart // EDGE_BLOCK) * EDGE_BLOCK
    nblk = ((eend + EDGE_BLOCK - 1) // EDGE_BLOCK) * EDGE_BLOCK - bstart
    nblk = nblk // EDGE_BLOCK
    meta = jnp.concatenate(
        [jnp.stack([lo, hi, bstart, nblk], axis=1),
         jnp.zeros((32, 12), jnp.int32)], axis=1)

    z16 = jnp.zeros((NFEAT, 16), f32)
    one15 = jnp.concatenate([jnp.ones((1,), f32), jnp.zeros((15,), f32)])
    wlt1 = jnp.concatenate(
        [gat1_Wl.T[:, 0:32], z16, gat1_Wl.T[:, 32:64], z16], axis=1)
    blaug1 = jnp.concatenate(
        [gat1_bl[0:32], one15, gat1_bl[32:64], one15])[None]
    wrt1 = gat1_Wr.T
    br1 = gat1_br[None]
    cfg1 = jnp.ones((16,), f32)
    cfg0 = jnp.zeros((16,), f32)

    zrow8 = jnp.zeros((8, FP), f32)
    xl1, xr1 = _proj1(x, wlt1, blaug1, wrt1, br1)
    p1 = _gat_edges()(jnp.concatenate([xl1, zrow8]), xr1, src, dst,
                      gat1_att, cfg1, meta)

    z64_16 = jnp.zeros((H1, 16), f32)
    wl23 = jnp.concatenate([gat2_Wl.T, z64_16, gat3_Wl.T, z64_16], axis=1)
    bl23 = jnp.concatenate([gat2_bl, one15, gat3_bl, one15])[None]
    wr23 = jnp.concatenate([gat2_Wr.T, gat3_Wr.T], axis=1)
    br23 = jnp.concatenate([gat2_br, gat3_br])[None]
    att23 = jnp.concatenate([gat2_att, gat3_att])

    xl23, xr23 = _mid(p1, bn1_g[None], bn1_b[None], gat1_b[None],
                      wl23, bl23, wr23, br23)
    p23 = _gat_edges()(jnp.concatenate([xl23, zrow8]), xr23, src, dst,
                       att23, cfg0, meta)

    mu, logvar, re_x = _dec(p23, gat2_b[None], gat3_b[None],
                            dec_W1.T, dec_b1[None], dec_bn_g[None],
                            dec_bn_b[None], dec_W2.T, dec_b2[None])
    readj = _readj(mu)
    return (mu, re_x, readj, mu, logvar)


# speedup vs baseline: 8.6211x; 1.2773x over previous
import functools

import jax
import jax.numpy as jnp
from jax import lax
from jax.experimental import pallas as pl
from jax.experimental.pallas import tpu as pltpu
from jax.experimental.pallas import tpu_sc as plsc

N = 10000
NFEAT = 128
NHID1 = 64
NHID2 = 32
H1 = 2 * NHID2
E = 160000
E_REAL = E + N
EDGE_BLOCK = 128
E_PAD = 172032
ROWS_PER_TILE = 624
TAIL_ROWS = N - 16 * ROWS_PER_TILE
TAIL_OFF = 16 * ROWS_PER_TILE
STAGE_ROWS = 48
NCHUNK = ROWS_PER_TILE // STAGE_ROWS
_BN_INV = float(1.0 / (1.0 + 1e-5) ** 0.5)

ROW_BLOCK = 1000
GRID_N = N // ROW_BLOCK


FP = 96
FR = 64
HEADS = ((0, 48, 0, 0, 32), (48, 48, 32, 32, 32))


NRANGE = 312
ACC_ROWS = 344
TRASH = 336


def _make_gat_edges(nc):
    mesh = plsc.VectorSubcoreMesh(core_axis_name="c", subcore_axis_name="s",
                                  num_cores=nc)

    @functools.partial(
        pl.kernel,
        mesh=mesh,
        compiler_params=pltpu.CompilerParams(needs_layout_passes=False,
                                             use_tc_tiling_on_sc=False),
        out_type=jax.ShapeDtypeStruct((N, FP), jnp.float32),
        scratch_types=[
            pltpu.VMEM((2, EDGE_BLOCK), jnp.int32),
            pltpu.VMEM((2, EDGE_BLOCK), jnp.int32),
            pltpu.VMEM((2, EDGE_BLOCK, FP), jnp.float32),
            pltpu.VMEM((2, EDGE_BLOCK, FR), jnp.float32),
            pltpu.VMEM((FR,), jnp.float32),
            pltpu.VMEM((16,), jnp.float32),
            pltpu.VMEM((16, 16), jnp.float32),
            pltpu.VMEM((32, 16), jnp.int32),
            pltpu.VMEM((ACC_ROWS, FP), jnp.float32),
            pltpu.SemaphoreType.DMA,
            pltpu.SemaphoreType.DMA,
            pltpu.SemaphoreType.DMA,
            pltpu.SemaphoreType.DMA,
        ],
    )
    def kern(xl_hbm, xr_hbm, src_hbm, dst_hbm, att_hbm, cfg_hbm, meta_hbm,
             out_hbm, idx_s2, idx_d2, gl2, gr2, att_v, cfg_v, dotbuf,
             meta_v, accbuf, sgl0, sgr0, sgl1, sgr1):
        cid = lax.axis_index("c")
        sid = lax.axis_index("s")
        wid = sid * nc + cid

        pltpu.sync_copy(att_hbm, att_v)
        pltpu.sync_copy(cfg_hbm, cfg_v)
        pltpu.sync_copy(meta_hbm, meta_v)
        att_regs = [att_v[pl.ds(16 * k, 16)] for k in range(FR // 16)]
        cvec = cfg_v[...]
        mv = meta_v[wid, :]
        lo = mv[0]
        hi = mv[1]
        bstart = mv[2]
        nblk = mv[3]

        zero16 = jnp.zeros((16,), jnp.float32)
        sems = ((sgl0, sgr0), (sgl1, sgr1))

        def fetch(b, slot):
            base = pl.multiple_of(bstart + b * EDGE_BLOCK, EDGE_BLOCK)
            pltpu.sync_copy(src_hbm.at[pl.ds(base, EDGE_BLOCK)],
                            idx_s2.at[slot])
            pltpu.sync_copy(dst_hbm.at[pl.ds(base, EDGE_BLOCK)],
                            idx_d2.at[slot])
            pltpu.async_copy(xl_hbm.at[idx_s2.at[slot]], gl2.at[slot],
                             sems[slot][0])
            pltpu.async_copy(xr_hbm.at[idx_d2.at[slot]], gr2.at[slot],
                             sems[slot][1])

        def wait_slot(slot):
            pltpu.make_async_copy(xl_hbm.at[pl.ds(0, EDGE_BLOCK)],
                                  gl2.at[slot], sems[slot][0]).wait()
            pltpu.make_async_copy(xr_hbm.at[pl.ds(0, EDGE_BLOCK)],
                                  gr2.at[slot], sems[slot][1]).wait()

        def compute_block(slot, carry):
            gl = gl2.at[slot]
            gr = gr2.at[slot]
            idx_d = idx_d2.at[slot]

            def t_body(t, tcarry):
                taccs, td_prev = tcarry
                e0 = t * 16
                rows = lax.iota(jnp.int32, 16)
                dvecs = []
                for (GO, GW, RO, AO, F) in HEADS:
                    for e_ in range(16):
                        acc = zero16
                        for k in range(F // 16):
                            a = gl[e0 + e_, pl.ds(GO + 16 * k, 16)]
                            r = gr[e0 + e_, pl.ds(RO + 16 * k, 16)]
                            m = a + r
                            m = jnp.maximum(m, 0.2 * m)
                            acc = acc + m * att_regs[AO // 16 + k]
                        dotbuf[e_, :] = acc
                    tot = zero16
                    for l in range(16):
                        tot = tot + plsc.load_gather(
                            dotbuf, [rows, jnp.full((16,), l, jnp.int32)])
                    dvecs.append(tot)
                wvec_a = jnp.exp(dvecs[0] + cvec * dvecs[1])
                wvec_b = jnp.exp(cvec * dvecs[0] + dvecs[1])
                dvec = idx_d[pl.ds(e0, 16)]
                for e_ in range(16):
                    e = e0 + e_
                    d = dvec[e_]
                    flush = d != td_prev
                    inr = jnp.logical_and(td_prev >= lo, td_prev < hi)
                    strow = jnp.where(jnp.logical_and(flush, inr),
                                      td_prev - lo, TRASH)
                    for j in range(FP // 16):
                        accbuf[strow, pl.ds(16 * j, 16)] = taccs[j]
                    keep = jnp.where(flush, 0.0, 1.0)
                    keepv = jnp.full((16,), keep)
                    wva = jnp.full((16,), wvec_a[e_])
                    wvb = jnp.full((16,), wvec_b[e_])
                    new = []
                    for j in range(FP // 16):
                        wv = wva if (16 * j) < 48 else wvb
                        contrib = gl[e, pl.ds(16 * j, 16)] * wv
                        new.append(taccs[j] * keepv + contrib)
                    taccs = new
                    td_prev = d
                return taccs, td_prev

            return lax.fori_loop(0, EDGE_BLOCK // 16, t_body, carry)

        fetch(0, 0)

        def pair_body(bb, carry):
            b0 = bb * 2
            fetch(b0 + 1, 1)
            wait_slot(0)
            carry = compute_block(0, carry)

            @pl.when(b0 + 2 < nblk)
            def _pf():
                fetch(b0 + 2, 0)

            wait_slot(1)
            carry = compute_block(1, carry)
            return carry

        accs0 = [zero16] * (FP // 16)
        accs, d_prev = lax.fori_loop(0, nblk // 2, pair_body,
                                     (accs0, jnp.int32(-1)))
        inr = jnp.logical_and(d_prev >= lo, d_prev < hi)
        strow = jnp.where(inr, d_prev - lo, TRASH)
        for j in range(FP // 16):
            accbuf[strow, pl.ds(16 * j, 16)] = accs[j]

        pltpu.sync_copy(accbuf.at[pl.ds(0, NRANGE)],
                        out_hbm.at[pl.ds(lo, NRANGE)])

        @pl.when(wid == 31)
        def _tail():
            pltpu.sync_copy(accbuf.at[pl.ds(NRANGE, 16)],
                            out_hbm.at[pl.ds(31 * NRANGE + NRANGE, 16)])

    return kern


NC = 2


@functools.lru_cache(maxsize=None)
def _gat_edges():
    return _make_gat_edges(NC)


def _proj1_body(x_ref, wlt_ref, blaug_ref, wrt_ref, br_ref, xl_out, xr_out):
    xb = x_ref[...]
    xl_out[...] = (jnp.dot(xb, wlt_ref[...], preferred_element_type=jnp.float32)
                   + blaug_ref[...])
    xr_out[...] = (jnp.dot(xb, wrt_ref[...], preferred_element_type=jnp.float32)
                   + br_ref[...])


def _proj1(x, wlt, blaug, wrt, br):
    return pl.pallas_call(
        _proj1_body,
        grid=(GRID_N,),
        in_specs=[
            pl.BlockSpec((ROW_BLOCK, NFEAT), lambda i: (i, 0)),
            pl.BlockSpec((NFEAT, FP), lambda i: (0, 0)),
            pl.BlockSpec((1, FP), lambda i: (0, 0)),
            pl.BlockSpec((NFEAT, H1), lambda i: (0, 0)),
            pl.BlockSpec((1, H1), lambda i: (0, 0)),
        ],
        out_specs=[
            pl.BlockSpec((ROW_BLOCK, FP), lambda i: (i, 0)),
            pl.BlockSpec((ROW_BLOCK, H1), lambda i: (i, 0)),
        ],
        out_shape=[
            jax.ShapeDtypeStruct((N, FP), jnp.float32),
            jax.ShapeDtypeStruct((N, H1), jnp.float32),
        ],
    )(x, wlt, blaug, wrt, br)


def _mid_body(p_ref, bn1g_ref, bn1b_ref, g1b_ref, wl23_ref, bl23_ref,
              wr23_ref, br23_ref, xl_out, xr_out):
    num = p_ref[...]
    den = num[:, 32:33]
    xl1 = jnp.concatenate([num[:, 0:32], num[:, 48:80]], axis=1)
    o = xl1 / (den + 1e-16) + g1b_ref[...]
    h = jnp.maximum(bn1g_ref[...] * (o * _BN_INV) + bn1b_ref[...], 0.0)
    xl_out[...] = (jnp.dot(h, wl23_ref[...], preferred_element_type=jnp.float32)
                   + bl23_ref[...])
    xr_out[...] = (jnp.dot(h, wr23_ref[...], preferred_element_type=jnp.float32)
                   + br23_ref[...])


def _mid(p, bn1g, bn1b, g1b, wl23, bl23, wr23, br23):
    return pl.pallas_call(
        _mid_body,
        grid=(GRID_N,),
        in_specs=[
            pl.BlockSpec((ROW_BLOCK, 96), lambda i: (i, 0)),
            pl.BlockSpec((1, H1), lambda i: (0, 0)),
            pl.BlockSpec((1, H1), lambda i: (0, 0)),
            pl.BlockSpec((1, H1), lambda i: (0, 0)),
            pl.BlockSpec((H1, 96), lambda i: (0, 0)),
            pl.BlockSpec((1, 96), lambda i: (0, 0)),
            pl.BlockSpec((H1, H1), lambda i: (0, 0)),
            pl.BlockSpec((1, H1), lambda i: (0, 0)),
        ],
        out_specs=[
            pl.BlockSpec((ROW_BLOCK, 96), lambda i: (i, 0)),
            pl.BlockSpec((ROW_BLOCK, H1), lambda i: (i, 0)),
        ],
        out_shape=[
            jax.ShapeDtypeStruct((N, 96), jnp.float32),
            jax.ShapeDtypeStruct((N, H1), jnp.float32),
        ],
    )(p, bn1g, bn1b, g1b, wl23, bl23, wr23, br23)


def _dec_body(p_ref, g2b_ref, g3b_ref, w1t_ref, db1_ref, bng_ref, bnb_ref,
              w2t_ref, db2_ref, mu_out, lv_out, rex_out):
    num = p_ref[...]
    mu = num[:, 0:32] / (num[:, 32:33] + 1e-16) + g2b_ref[...]
    lv = num[:, 48:80] / (num[:, 80:81] + 1e-16) + g3b_ref[...]
    mu_out[...] = mu
    lv_out[...] = lv
    d1 = jnp.dot(mu, w1t_ref[...], preferred_element_type=jnp.float32) + db1_ref[...]
    d1 = jnp.maximum(bng_ref[...] * (d1 * _BN_INV) + bnb_ref[...], 0.0)
    rex_out[...] = (jnp.dot(d1, w2t_ref[...], preferred_element_type=jnp.float32)
                    + db2_ref[...])


def _dec(p, g2b, g3b, w1t, db1, bng, bnb, w2t, db2):
    return pl.pallas_call(
        _dec_body,
        grid=(GRID_N,),
        in_specs=[
            pl.BlockSpec((ROW_BLOCK, 96), lambda i: (i, 0)),
            pl.BlockSpec((1, NHID2), lambda i: (0, 0)),
            pl.BlockSpec((1, NHID2), lambda i: (0, 0)),
            pl.BlockSpec((NHID2, NHID1), lambda i: (0, 0)),
            pl.BlockSpec((1, NHID1), lambda i: (0, 0)),
            pl.BlockSpec((1, NHID1), lambda i: (0, 0)),
            pl.BlockSpec((1, NHID1), lambda i: (0, 0)),
            pl.BlockSpec((NHID1, NFEAT), lambda i: (0, 0)),
            pl.BlockSpec((1, NFEAT), lambda i: (0, 0)),
        ],
        out_specs=[
            pl.BlockSpec((ROW_BLOCK, NHID2), lambda i: (i, 0)),
            pl.BlockSpec((ROW_BLOCK, NHID2), lambda i: (i, 0)),
            pl.BlockSpec((ROW_BLOCK, NFEAT), lambda i: (i, 0)),
        ],
        out_shape=[
            jax.ShapeDtypeStruct((N, NHID2), jnp.float32),
            jax.ShapeDtypeStruct((N, NHID2), jnp.float32),
            jax.ShapeDtypeStruct((N, NFEAT), jnp.float32),
        ],
    )(p, g2b, g3b, w1t, db1, bng, bnb, w2t, db2)


def _readj_body(rows_ref, full_ref, out_ref):
    logits = lax.dot_general(
        rows_ref[...], full_ref[...],
        dimension_numbers=(((1,), (1,)), ((), ())),
        preferred_element_type=jnp.float32)
    out_ref[...] = jax.nn.sigmoid(logits)


RJ_BLOCK = 400


def _readj(emb):
    return pl.pallas_call(
        _readj_body,
        grid=(N // RJ_BLOCK,),
        in_specs=[
            pl.BlockSpec((RJ_BLOCK, NHID2), lambda i: (i, 0)),
            pl.BlockSpec((N, NHID2), lambda i: (0, 0)),
        ],
        out_specs=pl.BlockSpec((RJ_BLOCK, N), lambda i: (i, 0)),
        out_shape=jax.ShapeDtypeStruct((N, N), jnp.float32),
    )(emb, emb)


def kernel(x, adj, gat1_Wl, gat1_bl, gat1_Wr, gat1_br, gat1_att, gat1_b,
           bn1_g, bn1_b, gat2_Wl, gat2_bl, gat2_Wr, gat2_br, gat2_att, gat2_b,
           gat3_Wl, gat3_bl, gat3_Wr, gat3_br, gat3_att, gat3_b,
           dec_W1, dec_b1, dec_bn_g, dec_bn_b, dec_W2, dec_b2):
    f32 = jnp.float32

    loop = jnp.arange(N, dtype=jnp.int32)
    src0 = jnp.concatenate(
        [adj[0], loop, jnp.full((E_PAD - E_REAL,), N, jnp.int32)])
    dst0 = jnp.concatenate(
        [adj[1], loop, jnp.zeros((E_PAD - E_REAL,), jnp.int32)])
    order = jnp.argsort(dst0)
    src = src0[order]
    dst = dst0[order]

    wids = jnp.arange(32, dtype=jnp.int32)
    lo = wids * NRANGE
    hi = jnp.where(wids == 31, N, lo + NRANGE).astype(jnp.int32)
    estart = jnp.searchsorted(dst, lo).astype(jnp.int32)
    eend = jnp.searchsorted(dst, hi).astype(jnp.int32)
    pair = 2 * EDGE_BLOCK
    bstart = (estart // pair) * pair
    nblk = (((eend + pair - 1) // pair) * pair - bstart) // EDGE_BLOCK
    meta = jnp.concatenate(
        [jnp.stack([lo, hi, bstart, nblk], axis=1),
         jnp.zeros((32, 12), jnp.int32)], axis=1)

    z16 = jnp.zeros((NFEAT, 16), f32)
    one15 = jnp.concatenate([jnp.ones((1,), f32), jnp.zeros((15,), f32)])
    wlt1 = jnp.concatenate(
        [gat1_Wl.T[:, 0:32], z16, gat1_Wl.T[:, 32:64], z16], axis=1)
    blaug1 = jnp.concatenate(
        [gat1_bl[0:32], one15, gat1_bl[32:64], one15])[None]
    wrt1 = gat1_Wr.T
    br1 = gat1_br[None]
    cfg1 = jnp.ones((16,), f32)
    cfg0 = jnp.zeros((16,), f32)

    zrow8 = jnp.zeros((8, FP), f32)
    xl1, xr1 = _proj1(x, wlt1, blaug1, wrt1, br1)
    p1 = _gat_edges()(jnp.concatenate([xl1, zrow8]), xr1, src, dst,
                      gat1_att, cfg1, meta)

    z64_16 = jnp.zeros((H1, 16), f32)
    wl23 = jnp.concatenate([gat2_Wl.T, z64_16, gat3_Wl.T, z64_16], axis=1)
    bl23 = jnp.concatenate([gat2_bl, one15, gat3_bl, one15])[None]
    wr23 = jnp.concatenate([gat2_Wr.T, gat3_Wr.T], axis=1)
    br23 = jnp.concatenate([gat2_br, gat3_br])[None]
    att23 = jnp.concatenate([gat2_att, gat3_att])

    xl23, xr23 = _mid(p1, bn1_g[None], bn1_b[None], gat1_b[None],
                      wl23, bl23, wr23, br23)
    p23 = _gat_edges()(jnp.concatenate([xl23, zrow8]), xr23, src, dst,
                       att23, cfg0, meta)

    mu, logvar, re_x = _dec(p23, gat2_b[None], gat3_b[None],
                            dec_W1.T, dec_b1[None], dec_bn_g[None],
                            dec_bn_b[None], dec_W2.T, dec_b2[None])
    readj = _readj(mu)
    return (mu, re_x, readj, mu, logvar)


# speedup vs baseline: 9.2201x; 1.0695x over previous
import functools

import jax
import jax.numpy as jnp
from jax import lax
from jax.experimental import pallas as pl
from jax.experimental.pallas import tpu as pltpu
from jax.experimental.pallas import tpu_sc as plsc

N = 10000
NFEAT = 128
NHID1 = 64
NHID2 = 32
H1 = 2 * NHID2
E = 160000
E_REAL = E + N
EDGE_BLOCK = 128
E_PAD = 172032
ROWS_PER_TILE = 624
TAIL_ROWS = N - 16 * ROWS_PER_TILE
TAIL_OFF = 16 * ROWS_PER_TILE
STAGE_ROWS = 48
NCHUNK = ROWS_PER_TILE // STAGE_ROWS
_BN_INV = float(1.0 / (1.0 + 1e-5) ** 0.5)

ROW_BLOCK = 1000
GRID_N = N // ROW_BLOCK


FP = 96
FR = 64
HEADS = ((0, 48, 0, 0, 32), (48, 48, 32, 32, 32))


NRANGE = 312
ACC_ROWS = 344
TRASH = 336


def _make_gat_edges(nc):
    mesh = plsc.VectorSubcoreMesh(core_axis_name="c", subcore_axis_name="s",
                                  num_cores=nc)

    @functools.partial(
        pl.kernel,
        mesh=mesh,
        compiler_params=pltpu.CompilerParams(needs_layout_passes=False,
                                             use_tc_tiling_on_sc=False),
        out_type=jax.ShapeDtypeStruct((N, FP), jnp.float32),
        scratch_types=[
            pltpu.VMEM((2, EDGE_BLOCK), jnp.int32),
            pltpu.VMEM((2, EDGE_BLOCK), jnp.int32),
            pltpu.VMEM((2, EDGE_BLOCK, FP), jnp.float32),
            pltpu.VMEM((2, EDGE_BLOCK, FR), jnp.float32),
            pltpu.VMEM((FR,), jnp.float32),
            pltpu.VMEM((16,), jnp.float32),
            pltpu.VMEM((16, 16), jnp.float32),
            pltpu.VMEM((32, 16), jnp.int32),
            pltpu.VMEM((ACC_ROWS, FP), jnp.float32),
            pltpu.SemaphoreType.DMA,
            pltpu.SemaphoreType.DMA,
            pltpu.SemaphoreType.DMA,
            pltpu.SemaphoreType.DMA,
        ],
    )
    def kern(xl_hbm, xr_hbm, src_hbm, dst_hbm, att_hbm, cfg_hbm, meta_hbm,
             out_hbm, idx_s2, idx_d2, gl2, gr2, att_v, cfg_v, dotbuf,
             meta_v, accbuf, sgl0, sgr0, sgl1, sgr1):
        cid = lax.axis_index("c")
        sid = lax.axis_index("s")
        wid = sid * nc + cid

        pltpu.sync_copy(att_hbm, att_v)
        pltpu.sync_copy(cfg_hbm, cfg_v)
        pltpu.sync_copy(meta_hbm, meta_v)
        att_regs = [att_v[pl.ds(16 * k, 16)] for k in range(FR // 16)]
        cvec = cfg_v[...]
        mv = meta_v[wid, :]
        lo = mv[0]
        hi = mv[1]
        bstart = mv[2]
        nblk = mv[3]

        zero16 = jnp.zeros((16,), jnp.float32)
        sems = ((sgl0, sgr0), (sgl1, sgr1))

        def fetch(b, slot):
            base = pl.multiple_of(bstart + b * EDGE_BLOCK, EDGE_BLOCK)
            pltpu.sync_copy(src_hbm.at[pl.ds(base, EDGE_BLOCK)],
                            idx_s2.at[slot])
            pltpu.sync_copy(dst_hbm.at[pl.ds(base, EDGE_BLOCK)],
                            idx_d2.at[slot])
            pltpu.async_copy(xl_hbm.at[idx_s2.at[slot]], gl2.at[slot],
                             sems[slot][0])
            pltpu.async_copy(xr_hbm.at[idx_d2.at[slot]], gr2.at[slot],
                             sems[slot][1])

        def wait_slot(slot):
            pltpu.make_async_copy(xl_hbm.at[pl.ds(0, EDGE_BLOCK)],
                                  gl2.at[slot], sems[slot][0]).wait()
            pltpu.make_async_copy(xr_hbm.at[pl.ds(0, EDGE_BLOCK)],
                                  gr2.at[slot], sems[slot][1]).wait()

        def compute_block(slot, carry):
            gl = gl2.at[slot]
            gr = gr2.at[slot]
            idx_d = idx_d2.at[slot]

            def t_body(t, tcarry):
                taccs, td_prev = tcarry
                e0 = t * 16
                rows = lax.iota(jnp.int32, 16)
                dvecs = []
                for (GO, GW, RO, AO, F) in HEADS:
                    for e_ in range(16):
                        acc = zero16
                        for k in range(F // 16):
                            a = gl[e0 + e_, pl.ds(GO + 16 * k, 16)]
                            r = gr[e0 + e_, pl.ds(RO + 16 * k, 16)]
                            m = a + r
                            m = jnp.maximum(m, 0.2 * m)
                            acc = acc + m * att_regs[AO // 16 + k]
                        dotbuf[e_, :] = acc
                    tot = zero16
                    for l in range(16):
                        tot = tot + plsc.load_gather(
                            dotbuf, [rows, jnp.full((16,), l, jnp.int32)])
                    dvecs.append(tot)
                wvec_a = jnp.exp(dvecs[0] + cvec * dvecs[1])
                wvec_b = jnp.exp(cvec * dvecs[0] + dvecs[1])
                dvec = idx_d[pl.ds(e0, 16)]
                for e_ in range(16):
                    e = e0 + e_
                    d = dvec[e_]
                    flush = d != td_prev
                    inr = jnp.logical_and(td_prev >= lo, td_prev < hi)
                    strow = jnp.where(inr, td_prev - lo, TRASH)
                    cur = list(taccs)

                    @pl.when(flush)
                    def _store():
                        for j in range(FP // 16):
                            accbuf[strow, pl.ds(16 * j, 16)] = cur[j]

                    keep = jnp.where(flush, 0.0, 1.0)
                    keepv = jnp.full((16,), keep)
                    wva = jnp.full((16,), wvec_a[e_])
                    wvb = jnp.full((16,), wvec_b[e_])
                    new = []
                    for j in range(FP // 16):
                        wv = wva if (16 * j) < 48 else wvb
                        contrib = gl[e, pl.ds(16 * j, 16)] * wv
                        new.append(taccs[j] * keepv + contrib)
                    taccs = new
                    td_prev = d
                return taccs, td_prev

            return lax.fori_loop(0, EDGE_BLOCK // 16, t_body, carry)

        fetch(0, 0)

        def pair_body(bb, carry):
            b0 = bb * 2
            fetch(b0 + 1, 1)
            wait_slot(0)
            carry = compute_block(0, carry)

            @pl.when(b0 + 2 < nblk)
            def _pf():
                fetch(b0 + 2, 0)

            wait_slot(1)
            carry = compute_block(1, carry)
            return carry

        accs0 = [zero16] * (FP // 16)
        accs, d_prev = lax.fori_loop(0, nblk // 2, pair_body,
                                     (accs0, jnp.int32(-1)))
        inr = jnp.logical_and(d_prev >= lo, d_prev < hi)
        strow = jnp.where(inr, d_prev - lo, TRASH)
        for j in range(FP // 16):
            accbuf[strow, pl.ds(16 * j, 16)] = accs[j]

        pltpu.sync_copy(accbuf.at[pl.ds(0, NRANGE)],
                        out_hbm.at[pl.ds(lo, NRANGE)])

        @pl.when(wid == 31)
        def _tail():
            pltpu.sync_copy(accbuf.at[pl.ds(NRANGE, 16)],
                            out_hbm.at[pl.ds(31 * NRANGE + NRANGE, 16)])

    return kern


NC = 2


@functools.lru_cache(maxsize=None)
def _gat_edges():
    return _make_gat_edges(NC)


def _proj1_body(x_ref, wlt_ref, blaug_ref, wrt_ref, br_ref, xl_out, xr_out):
    xb = x_ref[...]
    xl_out[...] = (jnp.dot(xb, wlt_ref[...], preferred_element_type=jnp.float32)
                   + blaug_ref[...])
    xr_out[...] = (jnp.dot(xb, wrt_ref[...], preferred_element_type=jnp.float32)
                   + br_ref[...])


def _proj1(x, wlt, blaug, wrt, br):
    return pl.pallas_call(
        _proj1_body,
        grid=(GRID_N,),
        in_specs=[
            pl.BlockSpec((ROW_BLOCK, NFEAT), lambda i: (i, 0)),
            pl.BlockSpec((NFEAT, FP), lambda i: (0, 0)),
            pl.BlockSpec((1, FP), lambda i: (0, 0)),
            pl.BlockSpec((NFEAT, H1), lambda i: (0, 0)),
            pl.BlockSpec((1, H1), lambda i: (0, 0)),
        ],
        out_specs=[
            pl.BlockSpec((ROW_BLOCK, FP), lambda i: (i, 0)),
            pl.BlockSpec((ROW_BLOCK, H1), lambda i: (i, 0)),
        ],
        out_shape=[
            jax.ShapeDtypeStruct((N, FP), jnp.float32),
            jax.ShapeDtypeStruct((N, H1), jnp.float32),
        ],
    )(x, wlt, blaug, wrt, br)


def _mid_body(p_ref, bn1g_ref, bn1b_ref, g1b_ref, wl23_ref, bl23_ref,
              wr23_ref, br23_ref, xl_out, xr_out):
    num = p_ref[...]
    den = num[:, 32:33]
    xl1 = jnp.concatenate([num[:, 0:32], num[:, 48:80]], axis=1)
    o = xl1 / (den + 1e-16) + g1b_ref[...]
    h = jnp.maximum(bn1g_ref[...] * (o * _BN_INV) + bn1b_ref[...], 0.0)
    xl_out[...] = (jnp.dot(h, wl23_ref[...], preferred_element_type=jnp.float32)
                   + bl23_ref[...])
    xr_out[...] = (jnp.dot(h, wr23_ref[...], preferred_element_type=jnp.float32)
                   + br23_ref[...])


def _mid(p, bn1g, bn1b, g1b, wl23, bl23, wr23, br23):
    return pl.pallas_call(
        _mid_body,
        grid=(GRID_N,),
        in_specs=[
            pl.BlockSpec((ROW_BLOCK, 96), lambda i: (i, 0)),
            pl.BlockSpec((1, H1), lambda i: (0, 0)),
            pl.BlockSpec((1, H1), lambda i: (0, 0)),
            pl.BlockSpec((1, H1), lambda i: (0, 0)),
            pl.BlockSpec((H1, 96), lambda i: (0, 0)),
            pl.BlockSpec((1, 96), lambda i: (0, 0)),
            pl.BlockSpec((H1, H1), lambda i: (0, 0)),
            pl.BlockSpec((1, H1), lambda i: (0, 0)),
        ],
        out_specs=[
            pl.BlockSpec((ROW_BLOCK, 96), lambda i: (i, 0)),
            pl.BlockSpec((ROW_BLOCK, H1), lambda i: (i, 0)),
        ],
        out_shape=[
            jax.ShapeDtypeStruct((N, 96), jnp.float32),
            jax.ShapeDtypeStruct((N, H1), jnp.float32),
        ],
    )(p, bn1g, bn1b, g1b, wl23, bl23, wr23, br23)


def _dec_body(p_ref, g2b_ref, g3b_ref, w1t_ref, db1_ref, bng_ref, bnb_ref,
              w2t_ref, db2_ref, mu_out, lv_out, rex_out):
    num = p_ref[...]
    mu = num[:, 0:32] / (num[:, 32:33] + 1e-16) + g2b_ref[...]
    lv = num[:, 48:80] / (num[:, 80:81] + 1e-16) + g3b_ref[...]
    mu_out[...] = mu
    lv_out[...] = lv
    d1 = jnp.dot(mu, w1t_ref[...], preferred_element_type=jnp.float32) + db1_ref[...]
    d1 = jnp.maximum(bng_ref[...] * (d1 * _BN_INV) + bnb_ref[...], 0.0)
    rex_out[...] = (jnp.dot(d1, w2t_ref[...], preferred_element_type=jnp.float32)
                    + db2_ref[...])


def _dec(p, g2b, g3b, w1t, db1, bng, bnb, w2t, db2):
    return pl.pallas_call(
        _dec_body,
        grid=(GRID_N,),
        in_specs=[
            pl.BlockSpec((ROW_BLOCK, 96), lambda i: (i, 0)),
            pl.BlockSpec((1, NHID2), lambda i: (0, 0)),
            pl.BlockSpec((1, NHID2), lambda i: (0, 0)),
            pl.BlockSpec((NHID2, NHID1), lambda i: (0, 0)),
            pl.BlockSpec((1, NHID1), lambda i: (0, 0)),
            pl.BlockSpec((1, NHID1), lambda i: (0, 0)),
            pl.BlockSpec((1, NHID1), lambda i: (0, 0)),
            pl.BlockSpec((NHID1, NFEAT), lambda i: (0, 0)),
            pl.BlockSpec((1, NFEAT), lambda i: (0, 0)),
        ],
        out_specs=[
            pl.BlockSpec((ROW_BLOCK, NHID2), lambda i: (i, 0)),
            pl.BlockSpec((ROW_BLOCK, NHID2), lambda i: (i, 0)),
            pl.BlockSpec((ROW_BLOCK, NFEAT), lambda i: (i, 0)),
        ],
        out_shape=[
            jax.ShapeDtypeStruct((N, NHID2), jnp.float32),
            jax.ShapeDtypeStruct((N, NHID2), jnp.float32),
            jax.ShapeDtypeStruct((N, NFEAT), jnp.float32),
        ],
    )(p, g2b, g3b, w1t, db1, bng, bnb, w2t, db2)


def _readj_body(rows_ref, full_ref, out_ref):
    logits = lax.dot_general(
        rows_ref[...], full_ref[...],
        dimension_numbers=(((1,), (1,)), ((), ())),
        preferred_element_type=jnp.float32)
    out_ref[...] = jax.nn.sigmoid(logits)


RJ_BLOCK = 400


def _readj(emb):
    return pl.pallas_call(
        _readj_body,
        grid=(N // RJ_BLOCK,),
        in_specs=[
            pl.BlockSpec((RJ_BLOCK, NHID2), lambda i: (i, 0)),
            pl.BlockSpec((N, NHID2), lambda i: (0, 0)),
        ],
        out_specs=pl.BlockSpec((RJ_BLOCK, N), lambda i: (i, 0)),
        out_shape=jax.ShapeDtypeStruct((N, N), jnp.float32),
    )(emb, emb)


def kernel(x, adj, gat1_Wl, gat1_bl, gat1_Wr, gat1_br, gat1_att, gat1_b,
           bn1_g, bn1_b, gat2_Wl, gat2_bl, gat2_Wr, gat2_br, gat2_att, gat2_b,
           gat3_Wl, gat3_bl, gat3_Wr, gat3_br, gat3_att, gat3_b,
           dec_W1, dec_b1, dec_bn_g, dec_bn_b, dec_W2, dec_b2):
    f32 = jnp.float32

    loop = jnp.arange(N, dtype=jnp.int32)
    src0 = jnp.concatenate(
        [adj[0], loop, jnp.full((E_PAD - E_REAL,), N, jnp.int32)])
    dst0 = jnp.concatenate(
        [adj[1], loop, jnp.zeros((E_PAD - E_REAL,), jnp.int32)])
    dst, src = lax.sort((dst0, src0), num_keys=1)

    wids = jnp.arange(32, dtype=jnp.int32)
    lo = wids * NRANGE
    hi = jnp.where(wids == 31, N, lo + NRANGE).astype(jnp.int32)
    estart = jnp.searchsorted(dst, lo).astype(jnp.int32)
    eend = jnp.searchsorted(dst, hi).astype(jnp.int32)
    pair = 2 * EDGE_BLOCK
    bstart = (estart // pair) * pair
    nblk = (((eend + pair - 1) // pair) * pair - bstart) // EDGE_BLOCK
    meta = jnp.concatenate(
        [jnp.stack([lo, hi, bstart, nblk], axis=1),
         jnp.zeros((32, 12), jnp.int32)], axis=1)

    z16 = jnp.zeros((NFEAT, 16), f32)
    one15 = jnp.concatenate([jnp.ones((1,), f32), jnp.zeros((15,), f32)])
    wlt1 = jnp.concatenate(
        [gat1_Wl.T[:, 0:32], z16, gat1_Wl.T[:, 32:64], z16], axis=1)
    blaug1 = jnp.concatenate(
        [gat1_bl[0:32], one15, gat1_bl[32:64], one15])[None]
    wrt1 = gat1_Wr.T
    br1 = gat1_br[None]
    cfg1 = jnp.ones((16,), f32)
    cfg0 = jnp.zeros((16,), f32)

    zrow8 = jnp.zeros((8, FP), f32)
    xl1, xr1 = _proj1(x, wlt1, blaug1, wrt1, br1)
    p1 = _gat_edges()(jnp.concatenate([xl1, zrow8]), xr1, src, dst,
                      gat1_att, cfg1, meta)

    z64_16 = jnp.zeros((H1, 16), f32)
    wl23 = jnp.concatenate([gat2_Wl.T, z64_16, gat3_Wl.T, z64_16], axis=1)
    bl23 = jnp.concatenate([gat2_bl, one15, gat3_bl, one15])[None]
    wr23 = jnp.concatenate([gat2_Wr.T, gat3_Wr.T], axis=1)
    br23 = jnp.concatenate([gat2_br, gat3_br])[None]
    att23 = jnp.concatenate([gat2_att, gat3_att])

    xl23, xr23 = _mid(p1, bn1_g[None], bn1_b[None], gat1_b[None],
                      wl23, bl23, wr23, br23)
    p23 = _gat_edges()(jnp.concatenate([xl23, zrow8]), xr23, src, dst,
                       att23, cfg0, meta)

    mu, logvar, re_x = _dec(p23, gat2_b[None], gat3_b[None],
                            dec_W1.T, dec_b1[None], dec_bn_g[None],
                            dec_bn_b[None], dec_W2.T, dec_b2[None])
    readj = _readj(mu)
    return (mu, re_x, readj, mu, logvar)


# speedup vs baseline: 9.3369x; 1.0127x over previous
import functools

import jax
import jax.numpy as jnp
from jax import lax
from jax.experimental import pallas as pl
from jax.experimental.pallas import tpu as pltpu
from jax.experimental.pallas import tpu_sc as plsc

N = 10000
NFEAT = 128
NHID1 = 64
NHID2 = 32
H1 = 2 * NHID2
E = 160000
E_REAL = E + N
EDGE_BLOCK = 128
E_PAD = 172032
ROWS_PER_TILE = 624
TAIL_ROWS = N - 16 * ROWS_PER_TILE
TAIL_OFF = 16 * ROWS_PER_TILE
STAGE_ROWS = 48
NCHUNK = ROWS_PER_TILE // STAGE_ROWS
_BN_INV = float(1.0 / (1.0 + 1e-5) ** 0.5)

ROW_BLOCK = 1000
GRID_N = N // ROW_BLOCK


FP = 96
FR = 64
HEADS = ((0, 48, 0, 0, 32), (48, 48, 32, 32, 32))


NRANGE = 312
ACC_ROWS = 344
TRASH = 336


def _make_gat_edges(nc):
    mesh = plsc.VectorSubcoreMesh(core_axis_name="c", subcore_axis_name="s",
                                  num_cores=nc)

    @functools.partial(
        pl.kernel,
        mesh=mesh,
        compiler_params=pltpu.CompilerParams(needs_layout_passes=False,
                                             use_tc_tiling_on_sc=False),
        out_type=jax.ShapeDtypeStruct((N, FP), jnp.float32),
        scratch_types=[
            pltpu.VMEM((2, EDGE_BLOCK), jnp.int32),
            pltpu.VMEM((2, EDGE_BLOCK), jnp.int32),
            pltpu.VMEM((2, EDGE_BLOCK, FP), jnp.float32),
            pltpu.VMEM((2, EDGE_BLOCK, FR), jnp.float32),
            pltpu.VMEM((FR,), jnp.float32),
            pltpu.VMEM((16,), jnp.float32),
            pltpu.VMEM((16, 16), jnp.float32),
            pltpu.VMEM((32, 16), jnp.int32),
            pltpu.VMEM((ACC_ROWS, FP), jnp.float32),
            pltpu.SemaphoreType.DMA,
            pltpu.SemaphoreType.DMA,
            pltpu.SemaphoreType.DMA,
            pltpu.SemaphoreType.DMA,
        ],
    )
    def kern(xl_hbm, xr_hbm, src_hbm, dst_hbm, att_hbm, cfg_hbm, meta_hbm,
             out_hbm, idx_s2, idx_d2, gl2, gr2, att_v, cfg_v, dotbuf,
             meta_v, accbuf, sgl0, sgr0, sgl1, sgr1):
        cid = lax.axis_index("c")
        sid = lax.axis_index("s")
        wid = sid * nc + cid

        pltpu.sync_copy(att_hbm, att_v)
        pltpu.sync_copy(cfg_hbm, cfg_v)
        pltpu.sync_copy(meta_hbm, meta_v)
        att_regs = [att_v[pl.ds(16 * k, 16)] for k in range(FR // 16)]
        cvec = cfg_v[...]
        mv = meta_v[wid, :]
        lo = mv[0]
        hi = mv[1]
        bstart = mv[2]
        nblk = mv[3]

        zero16 = jnp.zeros((16,), jnp.float32)
        sems = ((sgl0, sgr0), (sgl1, sgr1))

        def fetch(b, slot):
            base = pl.multiple_of(bstart + b * EDGE_BLOCK, EDGE_BLOCK)
            pltpu.sync_copy(src_hbm.at[pl.ds(base, EDGE_BLOCK)],
                            idx_s2.at[slot])
            pltpu.sync_copy(dst_hbm.at[pl.ds(base, EDGE_BLOCK)],
                            idx_d2.at[slot])
            pltpu.async_copy(xl_hbm.at[idx_s2.at[slot]], gl2.at[slot],
                             sems[slot][0])
            pltpu.async_copy(xr_hbm.at[idx_d2.at[slot]], gr2.at[slot],
                             sems[slot][1])

        def wait_slot(slot):
            pltpu.make_async_copy(xl_hbm.at[pl.ds(0, EDGE_BLOCK)],
                                  gl2.at[slot], sems[slot][0]).wait()
            pltpu.make_async_copy(xr_hbm.at[pl.ds(0, EDGE_BLOCK)],
                                  gr2.at[slot], sems[slot][1]).wait()

        def compute_block(slot, carry):
            gl = gl2.at[slot]
            gr = gr2.at[slot]
            idx_d = idx_d2.at[slot]

            def t_body(t, tcarry):
                taccs, td_prev = tcarry
                e0 = t * 16
                rows = lax.iota(jnp.int32, 16)
                dvecs = []
                for (GO, GW, RO, AO, F) in HEADS:
                    for e_ in range(16):
                        acc = zero16
                        for k in range(F // 16):
                            a = gl[e0 + e_, pl.ds(GO + 16 * k, 16)]
                            r = gr[e0 + e_, pl.ds(RO + 16 * k, 16)]
                            m = a + r
                            m = jnp.maximum(m, 0.2 * m)
                            acc = acc + m * att_regs[AO // 16 + k]
                        dotbuf[e_, :] = acc
                    tot = zero16
                    for l in range(16):
                        tot = tot + plsc.load_gather(
                            dotbuf, [rows, jnp.full((16,), l, jnp.int32)])
                    dvecs.append(tot)
                wvec_a = jnp.exp(dvecs[0] + cvec * dvecs[1])
                wvec_b = jnp.exp(cvec * dvecs[0] + dvecs[1])
                dvec = idx_d[pl.ds(e0, 16)]
                for e_ in range(16):
                    e = e0 + e_
                    d = dvec[e_]
                    flush = d != td_prev
                    inr = jnp.logical_and(td_prev >= lo, td_prev < hi)
                    strow = jnp.where(inr, td_prev - lo, TRASH)
                    cur = list(taccs)

                    @pl.when(flush)
                    def _store():
                        for j in range(FP // 16):
                            accbuf[strow, pl.ds(16 * j, 16)] = cur[j]

                    keep = jnp.where(flush, 0.0, 1.0)
                    keepv = jnp.full((16,), keep)
                    wva = jnp.full((16,), wvec_a[e_])
                    wvb = jnp.full((16,), wvec_b[e_])
                    new = []
                    for j in range(FP // 16):
                        wv = wva if (16 * j) < 48 else wvb
                        contrib = gl[e, pl.ds(16 * j, 16)] * wv
                        new.append(taccs[j] * keepv + contrib)
                    taccs = new
                    td_prev = d
                return taccs, td_prev

            return lax.fori_loop(0, EDGE_BLOCK // 16, t_body, carry)

        fetch(0, 0)

        def pair_body(bb, carry):
            b0 = bb * 2
            fetch(b0 + 1, 1)
            wait_slot(0)
            carry = compute_block(0, carry)

            @pl.when(b0 + 2 < nblk)
            def _pf():
                fetch(b0 + 2, 0)

            wait_slot(1)
            carry = compute_block(1, carry)
            return carry

        accs0 = [zero16] * (FP // 16)
        accs, d_prev = lax.fori_loop(0, nblk // 2, pair_body,
                                     (accs0, jnp.int32(-1)))
        inr = jnp.logical_and(d_prev >= lo, d_prev < hi)
        strow = jnp.where(inr, d_prev - lo, TRASH)
        for j in range(FP // 16):
            accbuf[strow, pl.ds(16 * j, 16)] = accs[j]

        pltpu.sync_copy(accbuf.at[pl.ds(0, NRANGE)],
                        out_hbm.at[pl.ds(lo, NRANGE)])

        @pl.when(wid == 31)
        def _tail():
            pltpu.sync_copy(accbuf.at[pl.ds(NRANGE, 16)],
                            out_hbm.at[pl.ds(31 * NRANGE + NRANGE, 16)])

    return kern


NC = 2


@functools.lru_cache(maxsize=None)
def _gat_edges():
    return _make_gat_edges(NC)


def _proj1_body(x_ref, wlt_ref, blaug_ref, wrt_ref, br_ref, xl_out, xr_out):
    xb = x_ref[...]
    xl_out[...] = (jnp.dot(xb, wlt_ref[...], preferred_element_type=jnp.float32)
                   + blaug_ref[...])
    xr_out[...] = (jnp.dot(xb, wrt_ref[...], preferred_element_type=jnp.float32)
                   + br_ref[...])


def _proj1(x, wlt, blaug, wrt, br):
    return pl.pallas_call(
        _proj1_body,
        grid=(GRID_N,),
        in_specs=[
            pl.BlockSpec((ROW_BLOCK, NFEAT), lambda i: (i, 0)),
            pl.BlockSpec((NFEAT, FP), lambda i: (0, 0)),
            pl.BlockSpec((1, FP), lambda i: (0, 0)),
            pl.BlockSpec((NFEAT, H1), lambda i: (0, 0)),
            pl.BlockSpec((1, H1), lambda i: (0, 0)),
        ],
        out_specs=[
            pl.BlockSpec((ROW_BLOCK, FP), lambda i: (i, 0)),
            pl.BlockSpec((ROW_BLOCK, H1), lambda i: (i, 0)),
        ],
        out_shape=[
            jax.ShapeDtypeStruct((N, FP), jnp.float32),
            jax.ShapeDtypeStruct((N, H1), jnp.float32),
        ],
    )(x, wlt, blaug, wrt, br)


def _mid_body(p_ref, bn1g_ref, bn1b_ref, g1b_ref, wl23_ref, bl23_ref,
              wr23_ref, br23_ref, xl_out, xr_out):
    num = p_ref[...]
    den = num[:, 32:33]
    xl1 = jnp.concatenate([num[:, 0:32], num[:, 48:80]], axis=1)
    o = xl1 / (den + 1e-16) + g1b_ref[...]
    h = jnp.maximum(bn1g_ref[...] * (o * _BN_INV) + bn1b_ref[...], 0.0)
    xl_out[...] = (jnp.dot(h, wl23_ref[...], preferred_element_type=jnp.float32)
                   + bl23_ref[...])
    xr_out[...] = (jnp.dot(h, wr23_ref[...], preferred_element_type=jnp.float32)
                   + br23_ref[...])


def _mid(p, bn1g, bn1b, g1b, wl23, bl23, wr23, br23):
    return pl.pallas_call(
        _mid_body,
        grid=(GRID_N,),
        in_specs=[
            pl.BlockSpec((ROW_BLOCK, 96), lambda i: (i, 0)),
            pl.BlockSpec((1, H1), lambda i: (0, 0)),
            pl.BlockSpec((1, H1), lambda i: (0, 0)),
            pl.BlockSpec((1, H1), lambda i: (0, 0)),
            pl.BlockSpec((H1, 96), lambda i: (0, 0)),
            pl.BlockSpec((1, 96), lambda i: (0, 0)),
            pl.BlockSpec((H1, H1), lambda i: (0, 0)),
            pl.BlockSpec((1, H1), lambda i: (0, 0)),
        ],
        out_specs=[
            pl.BlockSpec((ROW_BLOCK, 96), lambda i: (i, 0)),
            pl.BlockSpec((ROW_BLOCK, H1), lambda i: (i, 0)),
        ],
        out_shape=[
            jax.ShapeDtypeStruct((N, 96), jnp.float32),
            jax.ShapeDtypeStruct((N, H1), jnp.float32),
        ],
    )(p, bn1g, bn1b, g1b, wl23, bl23, wr23, br23)


def _dec_body(p_ref, g2b_ref, g3b_ref, w1t_ref, db1_ref, bng_ref, bnb_ref,
              w2t_ref, db2_ref, mu_out, lv_out, rex_out):
    num = p_ref[...]
    mu = num[:, 0:32] / (num[:, 32:33] + 1e-16) + g2b_ref[...]
    lv = num[:, 48:80] / (num[:, 80:81] + 1e-16) + g3b_ref[...]
    mu_out[...] = mu
    lv_out[...] = lv
    d1 = jnp.dot(mu, w1t_ref[...], preferred_element_type=jnp.float32) + db1_ref[...]
    d1 = jnp.maximum(bng_ref[...] * (d1 * _BN_INV) + bnb_ref[...], 0.0)
    rex_out[...] = (jnp.dot(d1, w2t_ref[...], preferred_element_type=jnp.float32)
                    + db2_ref[...])


def _dec(p, g2b, g3b, w1t, db1, bng, bnb, w2t, db2):
    return pl.pallas_call(
        _dec_body,
        grid=(GRID_N,),
        in_specs=[
            pl.BlockSpec((ROW_BLOCK, 96), lambda i: (i, 0)),
            pl.BlockSpec((1, NHID2), lambda i: (0, 0)),
            pl.BlockSpec((1, NHID2), lambda i: (0, 0)),
            pl.BlockSpec((NHID2, NHID1), lambda i: (0, 0)),
            pl.BlockSpec((1, NHID1), lambda i: (0, 0)),
            pl.BlockSpec((1, NHID1), lambda i: (0, 0)),
            pl.BlockSpec((1, NHID1), lambda i: (0, 0)),
            pl.BlockSpec((NHID1, NFEAT), lambda i: (0, 0)),
            pl.BlockSpec((1, NFEAT), lambda i: (0, 0)),
        ],
        out_specs=[
            pl.BlockSpec((ROW_BLOCK, NHID2), lambda i: (i, 0)),
            pl.BlockSpec((ROW_BLOCK, NHID2), lambda i: (i, 0)),
            pl.BlockSpec((ROW_BLOCK, NFEAT), lambda i: (i, 0)),
        ],
        out_shape=[
            jax.ShapeDtypeStruct((N, NHID2), jnp.float32),
            jax.ShapeDtypeStruct((N, NHID2), jnp.float32),
            jax.ShapeDtypeStruct((N, NFEAT), jnp.float32),
        ],
    )(p, g2b, g3b, w1t, db1, bng, bnb, w2t, db2)


def _readj_body(rows_ref, full_ref, out_ref):
    logits = lax.dot_general(
        rows_ref[...], full_ref[...],
        dimension_numbers=(((1,), (1,)), ((), ())),
        preferred_element_type=jnp.float32)
    out_ref[...] = 0.5 * jnp.tanh(0.5 * logits) + 0.5


RJ_BLOCK = 400


def _readj(emb):
    return pl.pallas_call(
        _readj_body,
        grid=(N // RJ_BLOCK,),
        in_specs=[
            pl.BlockSpec((RJ_BLOCK, NHID2), lambda i: (i, 0)),
            pl.BlockSpec((N, NHID2), lambda i: (0, 0)),
        ],
        out_specs=pl.BlockSpec((RJ_BLOCK, N), lambda i: (i, 0)),
        out_shape=jax.ShapeDtypeStruct((N, N), jnp.float32),
    )(emb, emb)


def kernel(x, adj, gat1_Wl, gat1_bl, gat1_Wr, gat1_br, gat1_att, gat1_b,
           bn1_g, bn1_b, gat2_Wl, gat2_bl, gat2_Wr, gat2_br, gat2_att, gat2_b,
           gat3_Wl, gat3_bl, gat3_Wr, gat3_br, gat3_att, gat3_b,
           dec_W1, dec_b1, dec_bn_g, dec_bn_b, dec_W2, dec_b2):
    f32 = jnp.float32

    loop = jnp.arange(N, dtype=jnp.int32)
    src0 = jnp.concatenate(
        [adj[0], loop, jnp.full((E_PAD - E_REAL,), N, jnp.int32)])
    dst0 = jnp.concatenate(
        [adj[1], loop, jnp.zeros((E_PAD - E_REAL,), jnp.int32)])
    dst, src = lax.sort((dst0, src0), num_keys=1)

    wids = jnp.arange(32, dtype=jnp.int32)
    lo = wids * NRANGE
    hi = jnp.where(wids == 31, N, lo + NRANGE).astype(jnp.int32)
    estart = jnp.searchsorted(dst, lo).astype(jnp.int32)
    eend = jnp.searchsorted(dst, hi).astype(jnp.int32)
    pair = 2 * EDGE_BLOCK
    bstart = (estart // pair) * pair
    nblk = (((eend + pair - 1) // pair) * pair - bstart) // EDGE_BLOCK
    meta = jnp.concatenate(
        [jnp.stack([lo, hi, bstart, nblk], axis=1),
         jnp.zeros((32, 12), jnp.int32)], axis=1)

    z16 = jnp.zeros((NFEAT, 16), f32)
    one15 = jnp.concatenate([jnp.ones((1,), f32), jnp.zeros((15,), f32)])
    wlt1 = jnp.concatenate(
        [gat1_Wl.T[:, 0:32], z16, gat1_Wl.T[:, 32:64], z16], axis=1)
    blaug1 = jnp.concatenate(
        [gat1_bl[0:32], one15, gat1_bl[32:64], one15])[None]
    wrt1 = gat1_Wr.T
    br1 = gat1_br[None]
    cfg1 = jnp.ones((16,), f32)
    cfg0 = jnp.zeros((16,), f32)

    zrow8 = jnp.zeros((8, FP), f32)
    xl1, xr1 = _proj1(x, wlt1, blaug1, wrt1, br1)
    p1 = _gat_edges()(jnp.concatenate([xl1, zrow8]), xr1, src, dst,
                      gat1_att, cfg1, meta)

    z64_16 = jnp.zeros((H1, 16), f32)
    wl23 = jnp.concatenate([gat2_Wl.T, z64_16, gat3_Wl.T, z64_16], axis=1)
    bl23 = jnp.concatenate([gat2_bl, one15, gat3_bl, one15])[None]
    wr23 = jnp.concatenate([gat2_Wr.T, gat3_Wr.T], axis=1)
    br23 = jnp.concatenate([gat2_br, gat3_br])[None]
    att23 = jnp.concatenate([gat2_att, gat3_att])

    xl23, xr23 = _mid(p1, bn1_g[None], bn1_b[None], gat1_b[None],
                      wl23, bl23, wr23, br23)
    p23 = _gat_edges()(jnp.concatenate([xl23, zrow8]), xr23, src, dst,
                       att23, cfg0, meta)

    mu, logvar, re_x = _dec(p23, gat2_b[None], gat3_b[None],
                            dec_W1.T, dec_b1[None], dec_bn_g[None],
                            dec_bn_b[None], dec_W2.T, dec_b2[None])
    readj = _readj(mu)
    return (mu, re_x, readj, mu, logvar)
